# Initial kernel scaffold; baseline (speedup 1.0000x reference)
import jax
import jax.numpy as jnp
from jax.experimental import pallas as pl


def kernel(inputs, edge_index, edge_attr, W_z, b_z, Wl_z, bl_z, W_r, b_r, Wl_r, bl_r, W_h, b_h, Wl_h, bl_h, W_reg, b_reg):
    raise NotImplementedError("write your pallas kernel here")


# speedup vs baseline: 33.0051x; 33.0051x over previous
import functools

import jax
import jax.numpy as jnp
from jax import lax
from jax.experimental import pallas as pl
from jax.experimental.pallas import tpu as pltpu
from jax.experimental.pallas import tpu_sc as plsc

_K = 80
_L = 16
_NBUF = 3

_BCAST_DNUMS = lax.GatherDimensionNumbers(
    offset_dims=(), collapsed_slice_dims=(0,), start_index_map=(0,))


def _lane_broadcast(vec16, k):
    idx = jnp.full((_L, 1), k, jnp.int32)
    return lax.gather(vec16, idx, _BCAST_DNUMS, (1,),
                      mode=lax.GatherScatterMode.PROMISE_IN_BOUNDS)


@functools.lru_cache(maxsize=None)
def _sc_aggregate(n, e, f_half):
    np_ = -(-n // 1280) * 1280
    ept = e // 16
    n_chunks = ept // _K
    n_rounds = -(-n_chunks // _NBUF)
    rows_per_tile = np_ // 16

    mesh = plsc.VectorSubcoreMesh(core_axis_name="c", subcore_axis_name="s")

    @functools.partial(
        pl.kernel,
        out_type=(
            jax.ShapeDtypeStruct((2, np_, f_half), jnp.float32),
            jax.ShapeDtypeStruct((n,), jnp.float32),
        ),
        mesh=mesh,
        compiler_params=pltpu.CompilerParams(needs_layout_passes=False),
        scratch_types=[
            pltpu.VMEM_SHARED((np_,), jnp.float32),
            pltpu.VMEM_SHARED((np_, f_half), jnp.float32),
            [pltpu.VMEM((_K, f_half), jnp.float32)] * _NBUF,
            [pltpu.VMEM((_K,), jnp.int32)] * _NBUF,
            [pltpu.VMEM((_K,), jnp.int32)] * _NBUF,
            [pltpu.VMEM((_K,), jnp.float32)] * _NBUF,
            [pltpu.VMEM((_K,), jnp.int32)] * _NBUF,
            [pltpu.VMEM((_K,), jnp.int32)] * _NBUF,
            [pltpu.VMEM((_K,), jnp.float32)] * _NBUF,
            pltpu.VMEM((np_,), jnp.float32),
            [pltpu.SemaphoreType.DMA] * _NBUF,
            [pltpu.SemaphoreType.DMA] * _NBUF,
            [pltpu.SemaphoreType.DMA] * _NBUF,
        ],
    )
    def sc_kernel(xcat, rowi, coli, ew, u_out, dis_out,
                  deg_sh, u_sh, rows, rowc, colc, ewc, rowg, colx, normb,
                  dis_v, isem, gsem, ssem):
        c = lax.axis_index("c")
        s = lax.axis_index("s")
        nvec = f_half // _L
        base_row = s * rows_per_tile
        ebase = s * ept

        z16 = jnp.zeros((_L,), jnp.float32)
        for i in range(_K):
            for j in range(nvec):
                rows[0][i, pl.ds(j * _L, _L)] = z16
        for g in range(_K // _L):
            normb[0][pl.ds(g * _L, _L)] = z16
        for j in range(rows_per_tile // _K):
            pltpu.sync_copy(normb[0], deg_sh.at[pl.ds(base_row + j * _K, _K)])
            pltpu.sync_copy(rows[0], u_sh.at[pl.ds(base_row + j * _K, _K), :])
        plsc.subcore_barrier()

        def deg_load(i, p):
            pltpu.async_copy(coli.at[pl.ds(ebase + i * _K, _K)], colc[p],
                             isem[p])
            pltpu.async_copy(ew.at[pl.ds(ebase + i * _K, _K)], ewc[p],
                             isem[p])

        def deg_wait(i, p):
            pltpu.make_async_copy(coli.at[pl.ds(ebase + i * _K, _K)], colc[p],
                                  isem[p]).wait()
            pltpu.make_async_copy(ew.at[pl.ds(ebase + i * _K, _K)], ewc[p],
                                  isem[p]).wait()

        for p in range(_NBUF):
            deg_load(p, p)

        @pl.loop(0, n_rounds)
        def _deg(j):
            for p in range(_NBUF):
                i = j * _NBUF + p

                @pl.when(i < n_chunks)
                def _():
                    deg_wait(i, p)
                    pltpu.sync_copy(ewc[p], deg_sh.at[colc[p]], add=True)

                @pl.when(i + _NBUF < n_chunks)
                def _():
                    deg_load(i + _NBUF, p)

        plsc.subcore_barrier()

        pltpu.sync_copy(deg_sh, dis_v)

        @pl.loop(0, np_ // _L)
        def _newton(i):
            d = dis_v[pl.ds(i * _L, _L)] + 1.0
            ok = d > 0.0
            bits = lax.bitcast_convert_type(d, jnp.int32)
            y = lax.bitcast_convert_type(jnp.int32(0x5F3759DF) - (bits >> 1),
                                         jnp.float32)
            h = d * 0.5
            y = y * (1.5 - h * y * y)
            y = y * (1.5 - h * y * y)
            y = y * (1.5 - h * y * y)
            dis_v[pl.ds(i * _L, _L)] = jnp.where(ok, y, 0.0)

        @pl.when(jnp.logical_and(c == 0, s == 0))
        def _():
            pltpu.sync_copy(dis_v.at[pl.ds(0, n)], dis_out)

        off = c * n

        def stage0(i, p):
            pltpu.async_copy(rowi.at[pl.ds(ebase + i * _K, _K)], rowc[p],
                             isem[p])
            pltpu.async_copy(coli.at[pl.ds(ebase + i * _K, _K)], colc[p],
                             isem[p])
            pltpu.async_copy(ew.at[pl.ds(ebase + i * _K, _K)], ewc[p],
                             isem[p])

        def stage1(i, p, wait_scatter=True):
            if wait_scatter:
                pltpu.make_async_copy(rows[p], u_sh.at[colx[p]],
                                      ssem[p]).wait()
            pltpu.make_async_copy(rowi.at[pl.ds(ebase + i * _K, _K)], rowc[p],
                                  isem[p]).wait()
            pltpu.make_async_copy(coli.at[pl.ds(ebase + i * _K, _K)], colc[p],
                                  isem[p]).wait()
            pltpu.make_async_copy(ew.at[pl.ds(ebase + i * _K, _K)], ewc[p],
                                  isem[p]).wait()
            for g in range(_K // _L):
                bsl = pl.ds(g * _L, _L)
                r16 = rowc[p][bsl]
                c16 = colc[p][bsl]
                dr = plsc.load_gather(dis_v, [r16])
                dc = plsc.load_gather(dis_v, [c16])
                normb[p][bsl] = dr * ewc[p][bsl] * dc
                rowg[p][bsl] = r16 + off
                colx[p][bsl] = c16
            pltpu.async_copy(xcat.at[rowg[p]], rows[p], gsem[p])

        def stage2(p):
            pltpu.make_async_copy(xcat.at[rowg[p]], rows[p], gsem[p]).wait()

            @pl.loop(0, _K // _L)
            def _scale(g):
                n16 = normb[p][pl.ds(g * _L, _L)]
                for kk in range(_L):
                    k = g * _L + kk
                    nb = _lane_broadcast(n16, kk)
                    for j in range(nvec):
                        sl = pl.ds(j * _L, _L)
                        rows[p][k, sl] = rows[p][k, sl] * nb

            pltpu.async_copy(rows[p], u_sh.at[colx[p]], ssem[p], add=True)

        for p in range(_NBUF):
            stage0(p, p)
        for p in range(_NBUF):
            stage1(p, p, wait_scatter=False)
            if _NBUF + p < n_chunks:
                stage0(_NBUF + p, p)

        @pl.loop(0, n_rounds)
        def _ring(j):
            for p in range(_NBUF):
                i2 = j * _NBUF + p
                i1 = i2 + _NBUF
                i0 = i2 + 2 * _NBUF

                @pl.when(i2 < n_chunks)
                def _():
                    stage2(p)

                @pl.when(i1 < n_chunks)
                def _():
                    stage1(i1, p)

                @pl.when(i0 < n_chunks)
                def _():
                    stage0(i0, p)

        for p in range(_NBUF):
            pltpu.make_async_copy(rows[p], u_sh.at[colx[p]], ssem[p]).wait()

        plsc.subcore_barrier()

        pltpu.sync_copy(u_sh.at[pl.ds(base_row, rows_per_tile), :],
                        u_out.at[c, pl.ds(base_row, rows_per_tile), :])

    return sc_kernel


@functools.lru_cache(maxsize=None)
def _dense(n, f_in, hid, f_out, blk):
    def body(x_ref, u0_ref, u1_ref, dis_ref,
             wz_ref, wh_ref, wlz_ref, wlh_ref, wreg_ref,
             bz_ref, bh_ref, blz_ref, blh_ref, breg_ref, o_ref):
        d = dis_ref[...]
        y = jnp.concatenate([u0_ref[...], u1_ref[...]], axis=1)
        y = y + (d * d) * x_ref[...]
        hi = jax.lax.Precision.HIGHEST
        cz = jnp.dot(y, wz_ref[...], preferred_element_type=jnp.float32,
                     precision=hi) + bz_ref[...]
        ch = jnp.dot(y, wh_ref[...], preferred_element_type=jnp.float32,
                     precision=hi) + bh_ref[...]
        z = jax.nn.sigmoid(jnp.dot(cz, wlz_ref[...],
                                   preferred_element_type=jnp.float32,
                                   precision=hi) + blz_ref[...])
        ht = jnp.tanh(jnp.dot(ch, wlh_ref[...],
                              preferred_element_type=jnp.float32,
                              precision=hi) + blh_ref[...])
        o_ref[...] = jnp.dot((1.0 - z) * ht, wreg_ref[...],
                             preferred_element_type=jnp.float32,
                             precision=hi) + breg_ref[...]

    fh = f_in // 2
    row_blk = lambda w: pl.BlockSpec((blk, w), lambda i: (i, 0))
    full = lambda a, b: pl.BlockSpec((a, b), lambda i: (0, 0))
    return pl.pallas_call(
        body,
        grid=(n // blk,),
        in_specs=[
            row_blk(f_in), row_blk(fh), row_blk(fh), row_blk(1),
            full(f_in, hid), full(f_in, hid), full(hid, hid), full(hid, hid),
            full(hid, f_out),
            full(1, hid), full(1, hid), full(1, hid), full(1, hid),
            full(1, f_out),
        ],
        out_specs=row_blk(f_out),
        out_shape=jax.ShapeDtypeStruct((n, f_out), jnp.float32),
    )


def kernel(inputs, edge_index, edge_attr,
           W_z, b_z, Wl_z, bl_z,
           W_r, b_r, Wl_r, bl_r,
           W_h, b_h, Wl_h, bl_h,
           W_reg, b_reg):
    n, f_in = inputs.shape
    hid = W_z.shape[1]
    e = edge_attr.shape[0]
    f_half = f_in // 2

    row = edge_index[0]
    col = edge_index[1]
    xcat = jnp.concatenate([inputs[:, :f_half], inputs[:, f_half:]], axis=0)

    u_pad, dis = _sc_aggregate(n, e, f_half)(xcat, row, col, edge_attr)
    u0 = u_pad[0, :n, :]
    u1 = u_pad[1, :n, :]

    blk = 400 if n % 400 == 0 else 200 if n % 200 == 0 else 8
    out = _dense(n, f_in, hid, W_reg.shape[1], blk)(
        inputs, u0, u1, dis[:, None],
        W_z, W_h, Wl_z[:hid], Wl_h[:hid], W_reg,
        b_z[None, :], b_h[None, :], bl_z[None, :], bl_h[None, :],
        b_reg[None, :])
    return out


# speedup vs baseline: 48.7475x; 1.4770x over previous
import functools

import jax
import jax.numpy as jnp
from jax import lax
from jax.experimental import pallas as pl
from jax.experimental.pallas import tpu as pltpu
from jax.experimental.pallas import tpu_sc as plsc

_K = 80
_L = 16
_NBUF = 3

_BCAST_DNUMS = lax.GatherDimensionNumbers(
    offset_dims=(), collapsed_slice_dims=(0,), start_index_map=(0,))


def _lane_broadcast(vec16, k):
    idx = jnp.full((_L, 1), k, jnp.int32)
    return lax.gather(vec16, idx, _BCAST_DNUMS, (1,),
                      mode=lax.GatherScatterMode.PROMISE_IN_BOUNDS)


@functools.lru_cache(maxsize=None)
def _sc_aggregate(n, e, f_half):
    np_ = -(-n // 1280) * 1280
    ept = e // 16
    n_chunks = ept // _K
    n_rounds = -(-n_chunks // _NBUF)
    rows_per_tile = np_ // 16

    mesh = plsc.VectorSubcoreMesh(core_axis_name="c", subcore_axis_name="s")

    @functools.partial(
        pl.kernel,
        out_type=(
            jax.ShapeDtypeStruct((2, np_, f_half), jnp.float32),
            jax.ShapeDtypeStruct((n,), jnp.float32),
        ),
        mesh=mesh,
        compiler_params=pltpu.CompilerParams(needs_layout_passes=False),
        scratch_types=[
            pltpu.VMEM_SHARED((np_,), jnp.float32),
            pltpu.VMEM_SHARED((np_, f_half), jnp.float32),
            [pltpu.VMEM((_K, f_half), jnp.float32)] * _NBUF,
            [pltpu.VMEM((_K,), jnp.int32)] * _NBUF,
            [pltpu.VMEM((_K,), jnp.int32)] * _NBUF,
            [pltpu.VMEM((_K,), jnp.float32)] * _NBUF,
            [pltpu.VMEM((_K,), jnp.int32)] * _NBUF,
            [pltpu.VMEM((_K,), jnp.int32)] * _NBUF,
            [pltpu.VMEM((_K,), jnp.float32)] * _NBUF,
            pltpu.VMEM((np_,), jnp.float32),
            [pltpu.SemaphoreType.DMA] * _NBUF,
            [pltpu.SemaphoreType.DMA] * _NBUF,
            [pltpu.SemaphoreType.DMA] * _NBUF,
        ],
    )
    def sc_kernel(xcat, rowi, coli, ew, u_out, dis_out,
                  deg_sh, u_sh, rows, rowc, colc, ewc, rowg, colx, normb,
                  dis_v, isem, gsem, ssem):
        c = lax.axis_index("c")
        s = lax.axis_index("s")
        nvec = f_half // _L
        base_row = s * rows_per_tile
        ebase = s * ept

        z16 = jnp.zeros((_L,), jnp.float32)
        for i in range(_K):
            for j in range(nvec):
                rows[0][i, pl.ds(j * _L, _L)] = z16
        for g in range(_K // _L):
            normb[0][pl.ds(g * _L, _L)] = z16
        for j in range(rows_per_tile // _K):
            pltpu.sync_copy(normb[0], deg_sh.at[pl.ds(base_row + j * _K, _K)])
            pltpu.sync_copy(rows[0], u_sh.at[pl.ds(base_row + j * _K, _K), :])
        plsc.subcore_barrier()

        def deg_load(i, p):
            pltpu.async_copy(coli.at[pl.ds(ebase + i * _K, _K)], colc[p],
                             isem[p])
            pltpu.async_copy(ew.at[pl.ds(ebase + i * _K, _K)], ewc[p],
                             isem[p])

        def deg_wait(i, p):
            pltpu.make_async_copy(coli.at[pl.ds(ebase + i * _K, _K)], colc[p],
                                  isem[p]).wait()
            pltpu.make_async_copy(ew.at[pl.ds(ebase + i * _K, _K)], ewc[p],
                                  isem[p]).wait()

        for p in range(_NBUF):
            deg_load(p, p)

        @pl.loop(0, n_rounds)
        def _deg(j):
            for p in range(_NBUF):
                i = j * _NBUF + p

                @pl.when(i < n_chunks)
                def _():
                    deg_wait(i, p)
                    pltpu.sync_copy(ewc[p], deg_sh.at[colc[p]], add=True)

                @pl.when(i + _NBUF < n_chunks)
                def _():
                    deg_load(i + _NBUF, p)

        plsc.subcore_barrier()

        pltpu.sync_copy(deg_sh, dis_v)

        @pl.loop(0, np_ // _L)
        def _newton(i):
            d = dis_v[pl.ds(i * _L, _L)] + 1.0
            ok = d > 0.0
            bits = lax.bitcast_convert_type(d, jnp.int32)
            y = lax.bitcast_convert_type(jnp.int32(0x5F3759DF) - (bits >> 1),
                                         jnp.float32)
            h = d * 0.5
            y = y * (1.5 - h * y * y)
            y = y * (1.5 - h * y * y)
            y = y * (1.5 - h * y * y)
            dis_v[pl.ds(i * _L, _L)] = jnp.where(ok, y, 0.0)

        @pl.when(jnp.logical_and(c == 0, s == 0))
        def _():
            pltpu.sync_copy(dis_v.at[pl.ds(0, n)], dis_out)

        off = c * n

        def stage0(i, p):
            pltpu.async_copy(rowi.at[pl.ds(ebase + i * _K, _K)], rowc[p],
                             isem[p])
            pltpu.async_copy(coli.at[pl.ds(ebase + i * _K, _K)], colc[p],
                             isem[p])
            pltpu.async_copy(ew.at[pl.ds(ebase + i * _K, _K)], ewc[p],
                             isem[p])

        def stage1(i, p, wait_scatter=True):
            if wait_scatter:
                pltpu.make_async_copy(rows[p], u_sh.at[colx[p]],
                                      ssem[p]).wait()
            pltpu.make_async_copy(rowi.at[pl.ds(ebase + i * _K, _K)], rowc[p],
                                  isem[p]).wait()
            pltpu.make_async_copy(coli.at[pl.ds(ebase + i * _K, _K)], colc[p],
                                  isem[p]).wait()
            pltpu.make_async_copy(ew.at[pl.ds(ebase + i * _K, _K)], ewc[p],
                                  isem[p]).wait()
            for g in range(_K // _L):
                bsl = pl.ds(g * _L, _L)
                r16 = rowc[p][bsl]
                c16 = colc[p][bsl]
                dr = plsc.load_gather(dis_v, [r16])
                dc = plsc.load_gather(dis_v, [c16])
                normb[p][bsl] = dr * ewc[p][bsl] * dc
                rowg[p][bsl] = r16 + off
                colx[p][bsl] = c16
            pltpu.async_copy(xcat.at[rowg[p]], rows[p], gsem[p])

        def stage2(p):
            pltpu.make_async_copy(xcat.at[rowg[p]], rows[p], gsem[p]).wait()

            @pl.loop(0, _K // _L)
            def _scale(g):
                n16 = normb[p][pl.ds(g * _L, _L)]
                for kk in range(_L):
                    k = g * _L + kk
                    nb = _lane_broadcast(n16, kk)
                    for j in range(nvec):
                        sl = pl.ds(j * _L, _L)
                        rows[p][k, sl] = rows[p][k, sl] * nb

            pltpu.async_copy(rows[p], u_sh.at[colx[p]], ssem[p], add=True)

        for p in range(_NBUF):
            stage0(p, p)
        for p in range(_NBUF):
            stage1(p, p, wait_scatter=False)
            if _NBUF + p < n_chunks:
                stage0(_NBUF + p, p)

        @pl.loop(0, n_rounds)
        def _ring(j):
            for p in range(_NBUF):
                i2 = j * _NBUF + p
                i1 = i2 + _NBUF
                i0 = i2 + 2 * _NBUF

                @pl.when(i2 < n_chunks)
                def _():
                    stage2(p)

                @pl.when(i1 < n_chunks)
                def _():
                    stage1(i1, p)

                @pl.when(i0 < n_chunks)
                def _():
                    stage0(i0, p)

        for p in range(_NBUF):
            pltpu.make_async_copy(rows[p], u_sh.at[colx[p]], ssem[p]).wait()

        plsc.subcore_barrier()

        pltpu.sync_copy(u_sh.at[pl.ds(base_row, rows_per_tile), :],
                        u_out.at[c, pl.ds(base_row, rows_per_tile), :])

    return sc_kernel


@functools.lru_cache(maxsize=None)
def _dense(n, f_in, hid, f_out, blk, prec):
    def body(x_ref, u0_ref, u1_ref, dis_ref,
             wz_ref, wh_ref, wlz_ref, wlh_ref, wreg_ref,
             bz_ref, bh_ref, blz_ref, blh_ref, breg_ref, o_ref):
        d = dis_ref[...]
        y = jnp.concatenate([u0_ref[0], u1_ref[0]], axis=1)
        y = y + (d * d) * x_ref[...]
        cz = jnp.dot(y, wz_ref[...], preferred_element_type=jnp.float32,
                     precision=prec) + bz_ref[...]
        ch = jnp.dot(y, wh_ref[...], preferred_element_type=jnp.float32,
                     precision=prec) + bh_ref[...]
        z = jax.nn.sigmoid(jnp.dot(cz, wlz_ref[...],
                                   preferred_element_type=jnp.float32,
                                   precision=prec) + blz_ref[...])
        ht = jnp.tanh(jnp.dot(ch, wlh_ref[...],
                              preferred_element_type=jnp.float32,
                              precision=prec) + blh_ref[...])
        o_ref[...] = jnp.dot((1.0 - z) * ht, wreg_ref[...],
                             preferred_element_type=jnp.float32,
                             precision=prec) + breg_ref[...]

    fh = f_in // 2
    row_blk = lambda w: pl.BlockSpec((blk, w), lambda i: (i, 0))
    upad_blk = lambda h: pl.BlockSpec((1, blk, fh), lambda i: (h, i, 0))
    full = lambda a, b: pl.BlockSpec((a, b), lambda i: (0, 0))
    return pl.pallas_call(
        body,
        grid=(n // blk,),
        in_specs=[
            row_blk(f_in), upad_blk(0), upad_blk(1), row_blk(1),
            full(f_in, hid), full(f_in, hid), full(hid, hid), full(hid, hid),
            full(hid, f_out),
            full(1, hid), full(1, hid), full(1, hid), full(1, hid),
            full(1, f_out),
        ],
        out_specs=row_blk(f_out),
        out_shape=jax.ShapeDtypeStruct((n, f_out), jnp.float32),
    )


def kernel(inputs, edge_index, edge_attr,
           W_z, b_z, Wl_z, bl_z,
           W_r, b_r, Wl_r, bl_r,
           W_h, b_h, Wl_h, bl_h,
           W_reg, b_reg):
    n, f_in = inputs.shape
    hid = W_z.shape[1]
    e = edge_attr.shape[0]
    f_half = f_in // 2

    row = edge_index[0]
    col = edge_index[1]
    xcat = jnp.concatenate([inputs[:, :f_half], inputs[:, f_half:]], axis=0)

    u_pad, dis = _sc_aggregate(n, e, f_half)(xcat, row, col, edge_attr)

    blk = 400 if n % 400 == 0 else 200 if n % 200 == 0 else 8
    out = _dense(n, f_in, hid, W_reg.shape[1], blk, "default")(
        inputs, u_pad, u_pad, dis[:, None],
        W_z, W_h, Wl_z, Wl_h, W_reg,
        b_z[None, :], b_h[None, :], bl_z[None, :], bl_h[None, :],
        b_reg[None, :])
    return out


# speedup vs baseline: 50.3965x; 1.0338x over previous
import functools

import jax
import jax.numpy as jnp
from jax import lax
from jax.experimental import pallas as pl
from jax.experimental.pallas import tpu as pltpu
from jax.experimental.pallas import tpu_sc as plsc

_K = 80
_L = 16
_NBUF = 3

_BCAST_DNUMS = lax.GatherDimensionNumbers(
    offset_dims=(), collapsed_slice_dims=(0,), start_index_map=(0,))


def _lane_broadcast(vec16, k):
    idx = jnp.full((_L, 1), k, jnp.int32)
    return lax.gather(vec16, idx, _BCAST_DNUMS, (1,),
                      mode=lax.GatherScatterMode.PROMISE_IN_BOUNDS)


@functools.lru_cache(maxsize=None)
def _sc_aggregate(n, e, f_half):
    np_ = -(-n // 1280) * 1280
    ept = e // 16
    n_chunks = ept // _K
    n_rounds = -(-n_chunks // _NBUF)
    rows_per_tile = np_ // 16

    mesh = plsc.VectorSubcoreMesh(core_axis_name="c", subcore_axis_name="s")

    @functools.partial(
        pl.kernel,
        out_type=(
            jax.ShapeDtypeStruct((2, np_, f_half), jnp.float32),
            jax.ShapeDtypeStruct((n,), jnp.float32),
        ),
        mesh=mesh,
        compiler_params=pltpu.CompilerParams(needs_layout_passes=False),
        scratch_types=[
            pltpu.VMEM_SHARED((np_,), jnp.float32),
            pltpu.VMEM_SHARED((np_, f_half), jnp.float32),
            [pltpu.VMEM((_K, f_half), jnp.float32)] * _NBUF,
            [pltpu.VMEM((_K,), jnp.int32)] * _NBUF,
            [pltpu.VMEM((_K,), jnp.int32)] * _NBUF,
            [pltpu.VMEM((_K,), jnp.float32)] * _NBUF,
            [pltpu.VMEM((_K,), jnp.int32)] * _NBUF,
            [pltpu.VMEM((_K,), jnp.int32)] * _NBUF,
            [pltpu.VMEM((_K,), jnp.float32)] * _NBUF,
            pltpu.VMEM((np_,), jnp.float32),
            [pltpu.SemaphoreType.DMA] * _NBUF,
            [pltpu.SemaphoreType.DMA] * _NBUF,
            [pltpu.SemaphoreType.DMA] * _NBUF,
        ],
    )
    def sc_kernel(xcat, rowi, coli, ew, u_out, dis_out,
                  deg_sh, u_sh, rows, rowc, colc, ewc, rowg, colx, normb,
                  dis_v, isem, gsem, ssem):
        c = lax.axis_index("c")
        s = lax.axis_index("s")
        nvec = f_half // _L
        base_row = s * rows_per_tile
        ebase = s * ept

        z16 = jnp.zeros((_L,), jnp.float32)
        for i in range(_K):
            for j in range(nvec):
                rows[0][i, pl.ds(j * _L, _L)] = z16
        for g in range(_K // _L):
            normb[0][pl.ds(g * _L, _L)] = z16
        for j in range(rows_per_tile // _K):
            pltpu.sync_copy(normb[0], deg_sh.at[pl.ds(base_row + j * _K, _K)])
            pltpu.sync_copy(rows[0], u_sh.at[pl.ds(base_row + j * _K, _K), :])
        plsc.subcore_barrier()

        def deg_load(i, p):
            pltpu.async_copy(coli.at[pl.ds(ebase + i * _K, _K)], colc[p],
                             isem[p])
            pltpu.async_copy(ew.at[pl.ds(ebase + i * _K, _K)], ewc[p],
                             isem[p])

        def deg_wait(i, p):
            pltpu.make_async_copy(coli.at[pl.ds(ebase + i * _K, _K)], colc[p],
                                  isem[p]).wait()
            pltpu.make_async_copy(ew.at[pl.ds(ebase + i * _K, _K)], ewc[p],
                                  isem[p]).wait()

        for p in range(_NBUF):
            deg_load(p, p)

        @pl.loop(0, n_rounds)
        def _deg(j):
            for p in range(_NBUF):
                i = j * _NBUF + p

                @pl.when(i < n_chunks)
                def _():
                    deg_wait(i, p)
                    pltpu.sync_copy(ewc[p], deg_sh.at[colc[p]], add=True)

                @pl.when(i + _NBUF < n_chunks)
                def _():
                    deg_load(i + _NBUF, p)

        plsc.subcore_barrier()

        pltpu.sync_copy(deg_sh, dis_v)

        @pl.loop(0, np_ // _L)
        def _newton(i):
            d = dis_v[pl.ds(i * _L, _L)] + 1.0
            ok = d > 0.0
            bits = lax.bitcast_convert_type(d, jnp.int32)
            y = lax.bitcast_convert_type(jnp.int32(0x5F3759DF) - (bits >> 1),
                                         jnp.float32)
            h = d * 0.5
            y = y * (1.5 - h * y * y)
            y = y * (1.5 - h * y * y)
            y = y * (1.5 - h * y * y)
            dis_v[pl.ds(i * _L, _L)] = jnp.where(ok, y, 0.0)

        @pl.when(jnp.logical_and(c == 0, s == 0))
        def _():
            pltpu.sync_copy(dis_v.at[pl.ds(0, n)], dis_out)

        off = c * n

        def stage0(i, p):
            pltpu.async_copy(rowi.at[pl.ds(ebase + i * _K, _K)], rowc[p],
                             isem[p])
            pltpu.async_copy(coli.at[pl.ds(ebase + i * _K, _K)], colc[p],
                             isem[p])
            pltpu.async_copy(ew.at[pl.ds(ebase + i * _K, _K)], ewc[p],
                             isem[p])

        def stage1(i, p, wait_scatter=True):
            if wait_scatter:
                pltpu.make_async_copy(rows[p], u_sh.at[colx[p]],
                                      ssem[p]).wait()
            pltpu.make_async_copy(rowi.at[pl.ds(ebase + i * _K, _K)], rowc[p],
                                  isem[p]).wait()
            pltpu.make_async_copy(coli.at[pl.ds(ebase + i * _K, _K)], colc[p],
                                  isem[p]).wait()
            pltpu.make_async_copy(ew.at[pl.ds(ebase + i * _K, _K)], ewc[p],
                                  isem[p]).wait()
            for g in range(_K // _L):
                bsl = pl.ds(g * _L, _L)
                r16 = rowc[p][bsl]
                c16 = colc[p][bsl]
                dr = plsc.load_gather(dis_v, [r16])
                dc = plsc.load_gather(dis_v, [c16])
                normb[p][bsl] = dr * ewc[p][bsl] * dc
                rowg[p][bsl] = r16 + off
                colx[p][bsl] = c16
            pltpu.async_copy(xcat.at[rowg[p]], rows[p], gsem[p])

        def stage2(p):
            pltpu.make_async_copy(xcat.at[rowg[p]], rows[p], gsem[p]).wait()

            @pl.loop(0, _K // _L)
            def _scale(g):
                n16 = normb[p][pl.ds(g * _L, _L)]
                for kk in range(_L):
                    k = g * _L + kk
                    nb = _lane_broadcast(n16, kk)
                    for j in range(nvec):
                        sl = pl.ds(j * _L, _L)
                        rows[p][k, sl] = rows[p][k, sl] * nb

            pltpu.async_copy(rows[p], u_sh.at[colx[p]], ssem[p], add=True)

        for p in range(_NBUF):
            stage0(p, p)
        for p in range(_NBUF):
            stage1(p, p, wait_scatter=False)
            if _NBUF + p < n_chunks:
                stage0(_NBUF + p, p)

        @pl.loop(0, n_rounds)
        def _ring(j):
            for p in range(_NBUF):
                i2 = j * _NBUF + p
                i1 = i2 + _NBUF
                i0 = i2 + 2 * _NBUF

                @pl.when(i2 < n_chunks)
                def _():
                    stage2(p)

                @pl.when(i1 < n_chunks)
                def _():
                    stage1(i1, p)

                @pl.when(i0 < n_chunks)
                def _():
                    stage0(i0, p)

        for p in range(_NBUF):
            pltpu.make_async_copy(rows[p], u_sh.at[colx[p]], ssem[p]).wait()

        plsc.subcore_barrier()

        pltpu.sync_copy(u_sh.at[pl.ds(base_row, rows_per_tile), :],
                        u_out.at[c, pl.ds(base_row, rows_per_tile), :])

    return sc_kernel


@functools.lru_cache(maxsize=None)
def _dense(n, f_in, hid, f_out, blk, prec):
    def body(x_ref, u0_ref, u1_ref, dis_ref,
             wz_ref, wh_ref, wlz_ref, wlh_ref, wreg_ref,
             bz_ref, bh_ref, blz_ref, blh_ref, breg_ref, o_ref):
        d = dis_ref[...]
        y = jnp.concatenate([u0_ref[0], u1_ref[0]], axis=1)
        y = y + (d * d) * x_ref[...]
        cz = jnp.dot(y, wz_ref[...], preferred_element_type=jnp.float32,
                     precision=prec) + bz_ref[...]
        ch = jnp.dot(y, wh_ref[...], preferred_element_type=jnp.float32,
                     precision=prec) + bh_ref[...]
        z = jax.nn.sigmoid(jnp.dot(cz, wlz_ref[...],
                                   preferred_element_type=jnp.float32,
                                   precision=prec) + blz_ref[...])
        ht = jnp.tanh(jnp.dot(ch, wlh_ref[...],
                              preferred_element_type=jnp.float32,
                              precision=prec) + blh_ref[...])
        o_ref[...] = jnp.dot((1.0 - z) * ht, wreg_ref[...],
                             preferred_element_type=jnp.float32,
                             precision=prec) + breg_ref[...]

    fh = f_in // 2
    row_blk = lambda w: pl.BlockSpec((blk, w), lambda i: (i, 0))
    upad_blk = lambda h: pl.BlockSpec((1, blk, fh), lambda i: (h, i, 0))
    full = lambda a, b: pl.BlockSpec((a, b), lambda i: (0, 0))
    return pl.pallas_call(
        body,
        grid=(n // blk,),
        in_specs=[
            row_blk(f_in), upad_blk(0), upad_blk(1), row_blk(1),
            full(f_in, hid), full(f_in, hid), full(hid, hid), full(hid, hid),
            full(hid, f_out),
            full(1, hid), full(1, hid), full(1, hid), full(1, hid),
            full(1, f_out),
        ],
        out_specs=row_blk(f_out),
        out_shape=jax.ShapeDtypeStruct((n, f_out), jnp.float32),
    )


def kernel(inputs, edge_index, edge_attr,
           W_z, b_z, Wl_z, bl_z,
           W_r, b_r, Wl_r, bl_r,
           W_h, b_h, Wl_h, bl_h,
           W_reg, b_reg):
    n, f_in = inputs.shape
    hid = W_z.shape[1]
    e = edge_attr.shape[0]
    f_half = f_in // 2

    row = edge_index[0]
    col = edge_index[1]
    xcat = jnp.concatenate([inputs[:, :f_half], inputs[:, f_half:]], axis=0)

    u_pad, dis = _sc_aggregate(n, e, f_half)(xcat, row, col, edge_attr)

    blk = 2000 if n % 2000 == 0 else 400 if n % 400 == 0 else 8
    out = _dense(n, f_in, hid, W_reg.shape[1], blk, "default")(
        inputs, u_pad, u_pad, dis[:, None],
        W_z, W_h, Wl_z, Wl_h, W_reg,
        b_z[None, :], b_h[None, :], bl_z[None, :], bl_h[None, :],
        b_reg[None, :])
    return out
